# baseline (speedup 1.0000x reference)
import jax
import jax.numpy as jnp
from jax import lax
from jax.experimental import pallas as pl
from jax.experimental.pallas import tpu as pltpu

H0 = W0 = 64
C0, C1, C2, C3 = 3, 32, 64, 128
NOUT = 2

HW0 = H0 * W0
H1 = W1 = 32; HW1 = H1 * W1
H2 = W2 = 16; HW2 = H2 * W2
H3 = W3 = 8;  HW3 = H3 * W3

NIMG = 4

MARG1 = 128
SEG1 = HW0 + 2 * MARG1
MARG2 = 64
SEG2 = HW1 + 2 * MARG2
MARG3 = 32
SEG3 = HW2 + 2 * MARG3

assert MARG1 >= W0 + 1 and MARG2 >= W1 + 1 and MARG3 >= W2 + 1


def _cnn_kernel(x_ref, w1_ref, b1_ref, w2_ref, b2_ref, w3_ref, b3_ref,
                wfc_ref, bfc_ref, o_ref,
                xsh, pat1, ps1, a2sh, ps2, a3sh, ps3):
    f32 = jnp.float32
    bf16 = jnp.bfloat16

    xsh[...] = jnp.zeros(xsh.shape, f32)
    for i in range(NIMG):
        xsh[:, i * SEG1 + MARG1:i * SEG1 + MARG1 + HW0] = x_ref[i]

    colp = lax.broadcasted_iota(jnp.int32, (C0, HW0), 1) % W0
    for dy in range(3):
        for dx in range(3):
            t = dy * 3 + dx
            off = (dy - 1) * W0 + (dx - 1)
            ox = dx - 1
            for i in range(NIMG):
                base = i * SEG1 + MARG1 + off
                piece = xsh[:, base:base + HW0]
                if dx != 1:
                    piece = jnp.where((colp + ox >= 0) & (colp + ox < W0),
                                      piece, 0.0)
                pat1[C0 * t:C0 * (t + 1), i * HW0:(i + 1) * HW0] = piece

    out1 = jnp.dot(w1_ref[...], pat1[...],
                   preferred_element_type=f32)
    ps1[...] = jnp.transpose(out1)

    n1 = NIMG * HW0
    xm1 = jnp.maximum(ps1[pl.ds(0, n1 // 2, 2), :], ps1[pl.ds(1, n1 // 2, 2), :])
    r1 = xm1.reshape(NIMG * H0 // 2, 2 * W1, C1)
    pooled1 = jnp.maximum(r1[:, 0:W1, :], r1[:, W1:2 * W1, :]).reshape(
        NIMG * HW1, C1)
    act1 = jnp.maximum(pooled1 + b1_ref[...], 0.0)

    a2sh[...] = jnp.zeros(a2sh.shape, bf16)
    a2sh[:, MARG2:MARG2 + HW1, :] = act1.astype(bf16).reshape(NIMG, HW1, C1)

    col2 = lax.broadcasted_iota(jnp.int32, (NIMG * HW1, 1), 0) % W1
    acc2 = None
    for dy in range(3):
        for dx in range(3):
            t = dy * 3 + dx
            off = (dy - 1) * W1 + (dx - 1)
            ox = dx - 1
            piece = a2sh[:, MARG2 + off:MARG2 + off + HW1, :].reshape(
                NIMG * HW1, C1)
            if dx != 1:
                piece = jnp.where((col2 + ox >= 0) & (col2 + ox < W1),
                                  piece, jnp.bfloat16(0))
            d = jnp.dot(piece, w2_ref[t], preferred_element_type=f32)
            acc2 = d if acc2 is None else acc2 + d
    ps2[...] = acc2

    n2 = NIMG * HW1
    xm2 = jnp.maximum(ps2[pl.ds(0, n2 // 2, 2), :], ps2[pl.ds(1, n2 // 2, 2), :])
    r2 = xm2.reshape(NIMG * H1 // 2, 2 * W2, C2)
    pooled2 = jnp.maximum(r2[:, 0:W2, :], r2[:, W2:2 * W2, :]).reshape(
        NIMG * HW2, C2)
    act2 = jnp.maximum(pooled2 + b2_ref[...], 0.0)

    a3sh[...] = jnp.zeros(a3sh.shape, bf16)
    a3sh[:, MARG3:MARG3 + HW2, :] = act2.astype(bf16).reshape(NIMG, HW2, C2)

    col3 = lax.broadcasted_iota(jnp.int32, (NIMG * HW2, 1), 0) % W2
    acc3 = None
    for dy in range(3):
        for dx in range(3):
            t = dy * 3 + dx
            off = (dy - 1) * W2 + (dx - 1)
            ox = dx - 1
            piece = a3sh[:, MARG3 + off:MARG3 + off + HW2, :].reshape(
                NIMG * HW2, C2)
            if dx != 1:
                piece = jnp.where((col3 + ox >= 0) & (col3 + ox < W2),
                                  piece, jnp.bfloat16(0))
            d = jnp.dot(piece, w3_ref[t], preferred_element_type=f32)
            acc3 = d if acc3 is None else acc3 + d
    ps3[...] = acc3

    n3 = NIMG * HW2
    xm3 = jnp.maximum(ps3[pl.ds(0, n3 // 2, 2), :], ps3[pl.ds(1, n3 // 2, 2), :])
    r3 = xm3.reshape(NIMG * H2 // 2, 2 * W3, C3)
    pooled3 = jnp.maximum(r3[:, 0:W3, :], r3[:, W3:2 * W3, :]).reshape(
        NIMG * HW3, C3)
    act3 = jnp.maximum(pooled3 + b3_ref[...], 0.0)

    a3r = act3.reshape(NIMG, HW3, C3)
    s0 = jnp.sum(jnp.sum(a3r * wfc_ref[0], axis=1), axis=1, keepdims=True)
    s1 = jnp.sum(jnp.sum(a3r * wfc_ref[1], axis=1), axis=1, keepdims=True)
    o_ref[0] = jnp.concatenate([s0, s1], axis=1) + bfc_ref[...]


@jax.jit
def _forward(x_nchw, params):
    B = x_nchw.shape[0]
    x_flat = x_nchw.reshape(B, C0, HW0)

    w1 = jnp.transpose(params["w1"].reshape(9 * C0, C1))
    w2 = params["w2"].reshape(9, C1, C2).astype(jnp.bfloat16)
    w3 = params["w3"].reshape(9, C2, C3).astype(jnp.bfloat16)
    wfc = jnp.transpose(params["w_fc"].reshape(NOUT, C3, H3, W3),
                        (0, 2, 3, 1)).reshape(NOUT, HW3, C3)
    b1 = params["b1"].reshape(1, C1)
    b2 = params["b2"].reshape(1, C2)
    b3 = params["b3"].reshape(1, C3)
    bfc = params["b_fc"].reshape(1, NOUT)

    nstep = B // NIMG
    out = pl.pallas_call(
        _cnn_kernel,
        out_shape=jax.ShapeDtypeStruct((nstep, NIMG, NOUT), jnp.float32),
        grid=(nstep,),
        in_specs=[
            pl.BlockSpec((NIMG, C0, HW0), lambda i: (i, 0, 0)),
            pl.BlockSpec((C1, 9 * C0), lambda i: (0, 0)),
            pl.BlockSpec((1, C1), lambda i: (0, 0)),
            pl.BlockSpec((9, C1, C2), lambda i: (0, 0, 0)),
            pl.BlockSpec((1, C2), lambda i: (0, 0)),
            pl.BlockSpec((9, C2, C3), lambda i: (0, 0, 0)),
            pl.BlockSpec((1, C3), lambda i: (0, 0)),
            pl.BlockSpec((NOUT, HW3, C3), lambda i: (0, 0, 0)),
            pl.BlockSpec((1, NOUT), lambda i: (0, 0)),
        ],
        out_specs=pl.BlockSpec((1, NIMG, NOUT), lambda i: (i, 0, 0)),
        scratch_shapes=[
            pltpu.VMEM((C0, NIMG * SEG1), jnp.float32),
            pltpu.VMEM((9 * C0, NIMG * HW0), jnp.float32),
            pltpu.VMEM((NIMG * HW0, C1), jnp.float32),
            pltpu.VMEM((NIMG, SEG2, C1), jnp.bfloat16),
            pltpu.VMEM((NIMG * HW1, C2), jnp.float32),
            pltpu.VMEM((NIMG, SEG3, C2), jnp.bfloat16),
            pltpu.VMEM((NIMG * HW2, C3), jnp.float32),
        ],
        compiler_params=pltpu.CompilerParams(
            dimension_semantics=("parallel",),
            vmem_limit_bytes=64 * 1024 * 1024),
    )(x_flat, w1, b1, w2, b2, w3, b3, wfc, bfc)
    return out.reshape(B, NOUT)


def kernel(x, w1, b1, w2, b2, w3, b3, w_fc, b_fc):
    params = {"w1": w1, "b1": b1, "w2": w2, "b2": b2,
              "w3": w3, "b3": b3, "w_fc": w_fc, "b_fc": b_fc}
    return _forward(x, params)


# speedup vs baseline: 1.0945x; 1.0945x over previous
import jax
import jax.numpy as jnp
from jax import lax
from jax.experimental import pallas as pl
from jax.experimental.pallas import tpu as pltpu

H0 = W0 = 64
C0, C1, C2, C3 = 3, 32, 64, 128
NOUT = 2

HW0 = H0 * W0
H1 = W1 = 32; HW1 = H1 * W1
H2 = W2 = 16; HW2 = H2 * W2
H3 = W3 = 8;  HW3 = H3 * W3

NIMG = 8

MARG1 = 128
SEG1 = HW0 + 2 * MARG1
MARG2 = 64
SEG2 = HW1 + 2 * MARG2
MARG3 = 32
SEG3 = HW2 + 2 * MARG3

assert MARG1 >= W0 + 1 and MARG2 >= W1 + 1 and MARG3 >= W2 + 1


def _cnn_kernel(x_ref, w1_ref, b1_ref, w2_ref, b2_ref, w3_ref, b3_ref,
                wfc_ref, bfc_ref, o_ref,
                xsh, pat1, ps1, a2m, a2c, a2p, ps2, a3m, a3c, a3p, ps3):
    f32 = jnp.float32
    bf16 = jnp.bfloat16

    xsh[...] = jnp.zeros(xsh.shape, f32)
    for i in range(NIMG):
        xsh[:, i * SEG1 + MARG1:i * SEG1 + MARG1 + HW0] = x_ref[i]

    colp = lax.broadcasted_iota(jnp.int32, (C0, HW0), 1) % W0
    for dy in range(3):
        for dx in range(3):
            t = dy * 3 + dx
            off = (dy - 1) * W0 + (dx - 1)
            ox = dx - 1
            for i in range(NIMG):
                base = i * SEG1 + MARG1 + off
                piece = xsh[:, base:base + HW0]
                if dx != 1:
                    piece = jnp.where((colp + ox >= 0) & (colp + ox < W0),
                                      piece, 0.0)
                pat1[C0 * t:C0 * (t + 1), i * HW0:(i + 1) * HW0] = piece

    out1 = jnp.dot(w1_ref[...], pat1[...],
                   preferred_element_type=f32)
    ps1[...] = jnp.transpose(out1)

    n1 = NIMG * HW0
    xm1 = jnp.maximum(ps1[pl.ds(0, n1 // 2, 2), :], ps1[pl.ds(1, n1 // 2, 2), :])
    r1 = xm1.reshape(NIMG * H0 // 2, 2 * W1, C1)
    pooled1 = jnp.maximum(r1[:, 0:W1, :], r1[:, W1:2 * W1, :]).reshape(
        NIMG * HW1, C1)
    act1 = jnp.maximum(pooled1 + b1_ref[...], 0.0)

    a2m[...] = jnp.zeros(a2m.shape, bf16)
    a2c[...] = jnp.zeros(a2c.shape, bf16)
    a2p[...] = jnp.zeros(a2p.shape, bf16)
    act1b = act1.astype(bf16)
    col2 = lax.broadcasted_iota(jnp.int32, (NIMG * HW1, 1), 0) % W1
    a2c[:, MARG2:MARG2 + HW1, :] = act1b.reshape(NIMG, HW1, C1)
    a2p[:, MARG2 - 1:MARG2 - 1 + HW1, :] = jnp.where(
        col2 != 0, act1b, jnp.bfloat16(0)).reshape(NIMG, HW1, C1)
    a2m[:, MARG2 + 1:MARG2 + 1 + HW1, :] = jnp.where(
        col2 != W1 - 1, act1b, jnp.bfloat16(0)).reshape(NIMG, HW1, C1)

    bufs2 = (a2m, a2c, a2p)
    acc2 = None
    for dy in range(3):
        for dx in range(3):
            t = dy * 3 + dx
            off = (dy - 1) * W1
            piece = bufs2[dx][:, MARG2 + off:MARG2 + off + HW1, :].reshape(
                NIMG * HW1, C1)
            d = jnp.dot(piece, w2_ref[t], preferred_element_type=f32)
            acc2 = d if acc2 is None else acc2 + d
    ps2[...] = acc2

    n2 = NIMG * HW1
    xm2 = jnp.maximum(ps2[pl.ds(0, n2 // 2, 2), :], ps2[pl.ds(1, n2 // 2, 2), :])
    r2 = xm2.reshape(NIMG * H1 // 2, 2 * W2, C2)
    pooled2 = jnp.maximum(r2[:, 0:W2, :], r2[:, W2:2 * W2, :]).reshape(
        NIMG * HW2, C2)
    act2 = jnp.maximum(pooled2 + b2_ref[...], 0.0)

    a3m[...] = jnp.zeros(a3m.shape, bf16)
    a3c[...] = jnp.zeros(a3c.shape, bf16)
    a3p[...] = jnp.zeros(a3p.shape, bf16)
    act2b = act2.astype(bf16)
    col3 = lax.broadcasted_iota(jnp.int32, (NIMG * HW2, 1), 0) % W2
    a3c[:, MARG3:MARG3 + HW2, :] = act2b.reshape(NIMG, HW2, C2)
    a3p[:, MARG3 - 1:MARG3 - 1 + HW2, :] = jnp.where(
        col3 != 0, act2b, jnp.bfloat16(0)).reshape(NIMG, HW2, C2)
    a3m[:, MARG3 + 1:MARG3 + 1 + HW2, :] = jnp.where(
        col3 != W2 - 1, act2b, jnp.bfloat16(0)).reshape(NIMG, HW2, C2)

    bufs3 = (a3m, a3c, a3p)
    acc3 = None
    for dy in range(3):
        for dx in range(3):
            t = dy * 3 + dx
            off = (dy - 1) * W2
            piece = bufs3[dx][:, MARG3 + off:MARG3 + off + HW2, :].reshape(
                NIMG * HW2, C2)
            d = jnp.dot(piece, w3_ref[t], preferred_element_type=f32)
            acc3 = d if acc3 is None else acc3 + d
    ps3[...] = acc3

    n3 = NIMG * HW2
    xm3 = jnp.maximum(ps3[pl.ds(0, n3 // 2, 2), :], ps3[pl.ds(1, n3 // 2, 2), :])
    r3 = xm3.reshape(NIMG * H2 // 2, 2 * W3, C3)
    pooled3 = jnp.maximum(r3[:, 0:W3, :], r3[:, W3:2 * W3, :]).reshape(
        NIMG * HW3, C3)
    act3 = jnp.maximum(pooled3 + b3_ref[...], 0.0)

    a3r = act3.reshape(NIMG, HW3, C3)
    s0 = jnp.sum(jnp.sum(a3r * wfc_ref[0], axis=1), axis=1, keepdims=True)
    s1 = jnp.sum(jnp.sum(a3r * wfc_ref[1], axis=1), axis=1, keepdims=True)
    o_ref[0] = jnp.concatenate([s0, s1], axis=1) + bfc_ref[...]


@jax.jit
def _forward(x_nchw, params):
    B = x_nchw.shape[0]
    x_flat = x_nchw.reshape(B, C0, HW0)

    w1 = jnp.transpose(params["w1"].reshape(9 * C0, C1))
    w2 = params["w2"].reshape(9, C1, C2).astype(jnp.bfloat16)
    w3 = params["w3"].reshape(9, C2, C3).astype(jnp.bfloat16)
    wfc = jnp.transpose(params["w_fc"].reshape(NOUT, C3, H3, W3),
                        (0, 2, 3, 1)).reshape(NOUT, HW3, C3)
    b1 = params["b1"].reshape(1, C1)
    b2 = params["b2"].reshape(1, C2)
    b3 = params["b3"].reshape(1, C3)
    bfc = params["b_fc"].reshape(1, NOUT)

    nstep = B // NIMG
    out = pl.pallas_call(
        _cnn_kernel,
        out_shape=jax.ShapeDtypeStruct((nstep, NIMG, NOUT), jnp.float32),
        grid=(nstep,),
        in_specs=[
            pl.BlockSpec((NIMG, C0, HW0), lambda i: (i, 0, 0)),
            pl.BlockSpec((C1, 9 * C0), lambda i: (0, 0)),
            pl.BlockSpec((1, C1), lambda i: (0, 0)),
            pl.BlockSpec((9, C1, C2), lambda i: (0, 0, 0)),
            pl.BlockSpec((1, C2), lambda i: (0, 0)),
            pl.BlockSpec((9, C2, C3), lambda i: (0, 0, 0)),
            pl.BlockSpec((1, C3), lambda i: (0, 0)),
            pl.BlockSpec((NOUT, HW3, C3), lambda i: (0, 0, 0)),
            pl.BlockSpec((1, NOUT), lambda i: (0, 0)),
        ],
        out_specs=pl.BlockSpec((1, NIMG, NOUT), lambda i: (i, 0, 0)),
        scratch_shapes=[
            pltpu.VMEM((C0, NIMG * SEG1), jnp.float32),
            pltpu.VMEM((9 * C0, NIMG * HW0), jnp.float32),
            pltpu.VMEM((NIMG * HW0, C1), jnp.float32),
            pltpu.VMEM((NIMG, SEG2, C1), jnp.bfloat16),
            pltpu.VMEM((NIMG, SEG2, C1), jnp.bfloat16),
            pltpu.VMEM((NIMG, SEG2, C1), jnp.bfloat16),
            pltpu.VMEM((NIMG * HW1, C2), jnp.float32),
            pltpu.VMEM((NIMG, SEG3, C2), jnp.bfloat16),
            pltpu.VMEM((NIMG, SEG3, C2), jnp.bfloat16),
            pltpu.VMEM((NIMG, SEG3, C2), jnp.bfloat16),
            pltpu.VMEM((NIMG * HW2, C3), jnp.float32),
        ],
        compiler_params=pltpu.CompilerParams(
            dimension_semantics=("parallel",),
            vmem_limit_bytes=64 * 1024 * 1024),
    )(x_flat, w1, b1, w2, b2, w3, b3, wfc, bfc)
    return out.reshape(B, NOUT)


def kernel(x, w1, b1, w2, b2, w3, b3, w_fc, b_fc):
    params = {"w1": w1, "b1": b1, "w2": w2, "b2": b2,
              "w3": w3, "b3": b3, "w_fc": w_fc, "b_fc": b_fc}
    return _forward(x, params)
